# Initial kernel scaffold; baseline (speedup 1.0000x reference)
import jax
import jax.numpy as jnp
from jax.experimental import pallas as pl


def kernel(inp_c, inp_n, edge_index, W1, b1, W2, b2, W3, b3, Wc, Wm1, bm1, Wm2, bm2, Wm3):
    raise NotImplementedError("write your pallas kernel here")


# speedup vs baseline: 3.4871x; 3.4871x over previous
import functools

import jax
import jax.numpy as jnp
from jax import lax
from jax.experimental import pallas as pl
from jax.experimental.pallas import tpu as pltpu
from jax.experimental.pallas import tpu_sc as plsc

N = 10000
E = 320000
H = 128

NW = 32
CH = 128
NCHUNK = 80
EPW = CH * NCHUNK
EPAD = NW * EPW
NT = 10112
RPS = NT // 16

NB = 1000
GRID = N // NB

_mesh = plsc.VectorSubcoreMesh(core_axis_name="c", subcore_axis_name="s",
                               num_cores=2, num_subcores=16)


@functools.partial(
    pl.kernel,
    out_type=jax.ShapeDtypeStruct((2, NT, H), jnp.float32),
    mesh=_mesh,
    scratch_types=[
        pltpu.VMEM((NCHUNK, CH), jnp.int32),
        pltpu.VMEM((NCHUNK, CH), jnp.int32),
        pltpu.VMEM((CH, H), jnp.float32),
        pltpu.VMEM_SHARED((NT, H), jnp.float32),
    ],
)
def _deg_kernel(src_hbm, dst_hbm, ones_src_hbm, ones_dst_hbm, zeros_hbm,
                out_hbm, src_v, dst_v, ones_v, table):
    c = lax.axis_index("c")
    s = lax.axis_index("s")
    wid = c * 16 + s
    pltpu.sync_copy(src_hbm.at[wid], src_v)
    pltpu.sync_copy(dst_hbm.at[wid], dst_v)
    pltpu.sync_copy(ones_src_hbm, ones_v)
    pltpu.sync_copy(zeros_hbm, table.at[pl.ds(s * RPS, RPS)])
    plsc.subcore_barrier()

    def body_s(j, carry):
        pltpu.sync_copy(ones_v, table.at[src_v.at[j]], add=True)
        return carry

    lax.fori_loop(0, NCHUNK, body_s, 0)
    pltpu.sync_copy(ones_dst_hbm, ones_v)

    def body_d(j, carry):
        pltpu.sync_copy(ones_v, table.at[dst_v.at[j]], add=True)
        return carry

    lax.fori_loop(0, NCHUNK, body_d, 0)
    plsc.subcore_barrier()
    pltpu.sync_copy(table.at[pl.ds(s * RPS, RPS)],
                    out_hbm.at[c].at[pl.ds(s * RPS, RPS)])


@functools.partial(
    pl.kernel,
    out_type=jax.ShapeDtypeStruct((2, NT, H), jnp.float32),
    mesh=_mesh,
    scratch_types=[
        pltpu.VMEM((NCHUNK, CH), jnp.int32),
        pltpu.VMEM((NCHUNK, CH), jnp.int32),
        pltpu.VMEM((CH, H), jnp.float32),
        pltpu.VMEM_SHARED((NT, H), jnp.float32),
        pltpu.SemaphoreType.DMA,
    ],
)
def _spmm_kernel(z_hbm, src_hbm, dst_hbm, zeros_hbm, out_hbm,
                 src_v, dst_v, rows_v, table, sem):
    c = lax.axis_index("c")
    s = lax.axis_index("s")
    wid = c * 16 + s
    pltpu.sync_copy(src_hbm.at[wid], src_v)
    pltpu.sync_copy(dst_hbm.at[wid], dst_v)
    pltpu.sync_copy(zeros_hbm, table.at[pl.ds(s * RPS, RPS)])
    plsc.subcore_barrier()

    def body(j, carry):
        pltpu.async_copy(z_hbm.at[src_v.at[j]], rows_v, sem).wait()
        pltpu.sync_copy(rows_v, table.at[dst_v.at[j]], add=True)
        return carry

    lax.fori_loop(0, NCHUNK, body, 0)
    plsc.subcore_barrier()
    pltpu.sync_copy(table.at[pl.ds(s * RPS, RPS)],
                    out_hbm.at[c].at[pl.ds(s * RPS, RPS)])


def _prep_body(deg_ref, xin_ref, z_ref, no_ref, ni_ref):
    dsrc = deg_ref[0, :, 0:1] + deg_ref[1, :, 0:1]
    ddst = deg_ref[0, :, H - 1:H] + deg_ref[1, :, H - 1:H]
    no = lax.rsqrt(jnp.maximum(dsrc, 1.0))
    ni = lax.rsqrt(jnp.maximum(ddst, 1.0))
    no_ref[...] = no
    ni_ref[...] = ni
    z_ref[...] = xin_ref[...] * no


def _prep_call(deg, xin):
    return pl.pallas_call(
        _prep_body,
        grid=(GRID,),
        in_specs=[
            pl.BlockSpec((2, NB, H), lambda i: (0, i, 0)),
            pl.BlockSpec((NB, H), lambda i: (i, 0)),
        ],
        out_specs=[
            pl.BlockSpec((NB, H), lambda i: (i, 0)),
            pl.BlockSpec((NB, 1), lambda i: (i, 0)),
            pl.BlockSpec((NB, 1), lambda i: (i, 0)),
        ],
        out_shape=[
            jax.ShapeDtypeStruct((N, H), jnp.float32),
            jax.ShapeDtypeStruct((N, 1), jnp.float32),
            jax.ShapeDtypeStruct((N, 1), jnp.float32),
        ],
    )(deg, xin)


def _layer_body(p_ref, ni_ref, no_ref, w_ref, b_ref,
                z_ref, rs_ref, rm_ref, sacc, macc):
    i = pl.program_id(0)
    agg = (p_ref[0] + p_ref[1]) * ni_ref[...]
    x = jnp.dot(agg, w_ref[...], preferred_element_type=jnp.float32, precision=lax.Precision.HIGHEST)
    x = jnp.maximum(x + b_ref[...], 0.0)
    z_ref[...] = x * no_ref[...]
    ps = jnp.sum(x, axis=0, keepdims=True)
    pm = jnp.max(x, axis=0, keepdims=True)

    @pl.when(i == 0)
    def _():
        sacc[...] = ps
        macc[...] = pm

    @pl.when(i > 0)
    def _():
        sacc[...] += ps
        macc[...] = jnp.maximum(macc[...], pm)

    @pl.when(i == pl.num_programs(0) - 1)
    def _():
        rs_ref[...] = sacc[...]
        rm_ref[...] = macc[...]


def _layer_call(partials, norm_in, norm_out, w, b):
    return pl.pallas_call(
        _layer_body,
        grid=(GRID,),
        in_specs=[
            pl.BlockSpec((2, NB, H), lambda i: (0, i, 0)),
            pl.BlockSpec((NB, 1), lambda i: (i, 0)),
            pl.BlockSpec((NB, 1), lambda i: (i, 0)),
            pl.BlockSpec((H, H), lambda i: (0, 0)),
            pl.BlockSpec((1, H), lambda i: (0, 0)),
        ],
        out_specs=[
            pl.BlockSpec((NB, H), lambda i: (i, 0)),
            pl.BlockSpec((1, H), lambda i: (0, 0)),
            pl.BlockSpec((1, H), lambda i: (0, 0)),
        ],
        out_shape=[
            jax.ShapeDtypeStruct((N, H), jnp.float32),
            jax.ShapeDtypeStruct((1, H), jnp.float32),
            jax.ShapeDtypeStruct((1, H), jnp.float32),
        ],
        scratch_shapes=[
            pltpu.VMEM((1, H), jnp.float32),
            pltpu.VMEM((1, H), jnp.float32),
        ],
    )(partials, norm_in, norm_out, w, b)


def _head_body(inpc_ref, wc_ref, rs1, rm1, rs2, rm2, rs3, rm3,
               w1c_ref, w1s_ref, w1m_ref, bm1_ref, wm2_ref, bm2_ref, wm3_ref,
               out_ref):
    embed = jnp.maximum(
        jnp.dot(inpc_ref[...], wc_ref[...], preferred_element_type=jnp.float32, precision=lax.Precision.HIGHEST),
        0.0)
    inv_n = 1.0 / N
    hs = (jnp.maximum(rs1[...] * inv_n, 0.0)
          + jnp.maximum(rs2[...] * inv_n, 0.0)
          + jnp.maximum(rs3[...] * inv_n, 0.0))
    hm = (jnp.maximum(rm1[...], 0.0) + jnp.maximum(rm2[...], 0.0)
          + jnp.maximum(rm3[...], 0.0))
    h = (jnp.dot(embed, w1c_ref[...], preferred_element_type=jnp.float32, precision=lax.Precision.HIGHEST)
         + jnp.dot(hs, w1s_ref[...], preferred_element_type=jnp.float32, precision=lax.Precision.HIGHEST)
         + jnp.dot(hm, w1m_ref[...], preferred_element_type=jnp.float32, precision=lax.Precision.HIGHEST)
         + bm1_ref[...])
    h = jnp.maximum(h, 0.0)
    h = jnp.maximum(
        jnp.dot(h, wm2_ref[...], preferred_element_type=jnp.float32, precision=lax.Precision.HIGHEST)
        + bm2_ref[...], 0.0)
    out_ref[...] = jnp.dot(h, wm3_ref[...], preferred_element_type=jnp.float32, precision=lax.Precision.HIGHEST)


def _head_call(inp_c, wc, rs1, rm1, rs2, rm2, rs3, rm3,
               w1c, w1s, w1m, bm1, wm2, bm2, wm3):
    return pl.pallas_call(
        _head_body,
        out_shape=jax.ShapeDtypeStruct((1, 1), jnp.float32),
    )(inp_c, wc, rs1, rm1, rs2, rm2, rs3, rm3,
      w1c, w1s, w1m, bm1, wm2, bm2, wm3)


def kernel(inp_c, inp_n, edge_index, W1, b1, W2, b2, W3, b3,
           Wc, Wm1, bm1, Wm2, bm2, Wm3):
    src = edge_index[0]
    dst = edge_index[1]
    pad = EPAD - E
    src_p = jnp.concatenate(
        [src, jnp.zeros((pad,), jnp.int32)]).reshape(NW, NCHUNK, CH)
    dst_p = jnp.concatenate(
        [dst, jnp.full((pad,), N, jnp.int32)]).reshape(NW, NCHUNK, CH)

    col = jnp.arange(H, dtype=jnp.float32)
    ones_src = jnp.tile(jnp.where(col < 64, 1.0, 0.0)[None, :], (CH, 1))
    ones_dst = jnp.tile(jnp.where(col < 64, 0.0, 1.0)[None, :], (CH, 1))
    zeros_h = jnp.zeros((RPS, H), jnp.float32)

    deg = _deg_kernel(src_p, dst_p, ones_src, ones_dst, zeros_h)
    z, norm_out, norm_in = _prep_call(deg, inp_n)

    readouts = []
    for w, b in ((W1, b1), (W2, b2), (W3, b3)):
        partials = _spmm_kernel(z, src_p, dst_p, zeros_h)
        z, rs, rm = _layer_call(partials, norm_in, norm_out,
                                w, b.reshape(1, H))
        readouts += [rs, rm]

    w1c = Wm1[:64]
    w1s = Wm1[64:64 + H]
    w1m = Wm1[64 + H:]
    return _head_call(inp_c, Wc, *readouts,
                      w1c, w1s, w1m, bm1.reshape(1, -1),
                      Wm2, bm2.reshape(1, -1), Wm3)


# speedup vs baseline: 3.7957x; 1.0885x over previous
import functools

import jax
import jax.numpy as jnp
from jax import lax
from jax.experimental import pallas as pl
from jax.experimental.pallas import tpu as pltpu
from jax.experimental.pallas import tpu_sc as plsc

N = 10000
E = 320000
H = 128

NW = 32
CH = 128
NCHUNK = 80
EPW = CH * NCHUNK
EPAD = NW * EPW
NT = 10112
RPS = NT // 16

NB = 1000
GRID = N // NB

_mesh = plsc.VectorSubcoreMesh(core_axis_name="c", subcore_axis_name="s",
                               num_cores=2, num_subcores=16)


@functools.partial(
    pl.kernel,
    out_type=jax.ShapeDtypeStruct((2, NT, H), jnp.float32),
    mesh=_mesh,
    scratch_types=[
        pltpu.VMEM((NCHUNK, CH), jnp.int32),
        pltpu.VMEM((NCHUNK, CH), jnp.int32),
        pltpu.VMEM((CH, H), jnp.float32),
        pltpu.VMEM_SHARED((NT, H), jnp.float32),
        pltpu.SemaphoreType.DMA,
    ],
)
def _deg_kernel(src_hbm, dst_hbm, ones_src_hbm, ones_dst_hbm, zeros_hbm,
                out_hbm, src_v, dst_v, ones_v, table, sems):
    c = lax.axis_index("c")
    s = lax.axis_index("s")
    wid = c * 16 + s
    pltpu.sync_copy(src_hbm.at[wid], src_v)
    pltpu.sync_copy(dst_hbm.at[wid], dst_v)
    pltpu.sync_copy(ones_src_hbm, ones_v)
    pltpu.sync_copy(zeros_hbm, table.at[pl.ds(s * RPS, RPS)])
    plsc.subcore_barrier()

    DEPTH = 4

    def scatter_pass(idx_v):
        def body(j, carry):
            pltpu.async_copy(ones_v, table.at[idx_v.at[j]], sems, add=True)

            @pl.when(j >= DEPTH)
            def _():
                pltpu.make_async_copy(ones_v, table.at[idx_v.at[j]],
                                      sems).wait()
            return carry

        lax.fori_loop(0, NCHUNK, body, 0)
        for _ in range(DEPTH):
            pltpu.make_async_copy(ones_v, table.at[idx_v.at[0]], sems).wait()

    scatter_pass(src_v)
    pltpu.sync_copy(ones_dst_hbm, ones_v)
    scatter_pass(dst_v)
    plsc.subcore_barrier()
    pltpu.sync_copy(table.at[pl.ds(s * RPS, RPS)],
                    out_hbm.at[c].at[pl.ds(s * RPS, RPS)])


HC = NCHUNK // 2


@functools.partial(
    pl.kernel,
    out_type=jax.ShapeDtypeStruct((2, NT, H), jnp.float32),
    mesh=_mesh,
    scratch_types=[
        pltpu.VMEM((HC, CH), jnp.int32),
        pltpu.VMEM((HC, CH), jnp.int32),
        pltpu.VMEM((2 * CH, H), jnp.float32),
        pltpu.VMEM_SHARED((NT, H), jnp.float32),
        pltpu.SemaphoreType.DMA,
        pltpu.SemaphoreType.DMA,
    ],
)
def _spmm_kernel(z_hbm, src_hbm, dst_hbm, zeros_hbm, out_hbm,
                 src_v, dst_v, rows_v, table, semg, sems):
    c = lax.axis_index("c")
    s = lax.axis_index("s")
    wid = c * 16 + s
    pltpu.sync_copy(zeros_hbm, table.at[pl.ds(s * RPS, RPS)])
    plsc.subcore_barrier()

    for h in range(2):
        pltpu.sync_copy(src_hbm.at[wid].at[pl.ds(h * HC, HC)], src_v)
        pltpu.sync_copy(dst_hbm.at[wid].at[pl.ds(h * HC, HC)], dst_v)
        pltpu.async_copy(z_hbm.at[src_v.at[0]], rows_v.at[pl.ds(0, CH)], semg)

        def body(j, carry):
            b = (j % 2) * CH
            buf = rows_v.at[pl.ds(b, CH)]
            pltpu.make_async_copy(z_hbm.at[src_v.at[j]], buf, semg).wait()
            pltpu.async_copy(buf, table.at[dst_v.at[j]], sems, add=True)

            @pl.when(j + 1 < HC)
            def _():
                nbuf = rows_v.at[pl.ds(CH - b, CH)]

                @pl.when(j >= 1)
                def _():
                    pltpu.make_async_copy(
                        nbuf, table.at[dst_v.at[j]], sems).wait()
                pltpu.async_copy(z_hbm.at[src_v.at[j + 1]], nbuf, semg)
            return carry

        lax.fori_loop(0, HC, body, 0)
        for _ in range(2):
            pltpu.make_async_copy(rows_v.at[pl.ds(0, CH)],
                                  table.at[dst_v.at[0]], sems).wait()

    plsc.subcore_barrier()
    pltpu.sync_copy(table.at[pl.ds(s * RPS, RPS)],
                    out_hbm.at[c].at[pl.ds(s * RPS, RPS)])


def _prep_body(deg_ref, xin_ref, z_ref, no_ref, ni_ref):
    dsrc = deg_ref[0, :, 0:1] + deg_ref[1, :, 0:1]
    ddst = deg_ref[0, :, H - 1:H] + deg_ref[1, :, H - 1:H]
    no = lax.rsqrt(jnp.maximum(dsrc, 1.0))
    ni = lax.rsqrt(jnp.maximum(ddst, 1.0))
    no_ref[...] = no
    ni_ref[...] = ni
    z_ref[...] = xin_ref[...] * no


def _prep_call(deg, xin):
    return pl.pallas_call(
        _prep_body,
        grid=(GRID,),
        in_specs=[
            pl.BlockSpec((2, NB, H), lambda i: (0, i, 0)),
            pl.BlockSpec((NB, H), lambda i: (i, 0)),
        ],
        out_specs=[
            pl.BlockSpec((NB, H), lambda i: (i, 0)),
            pl.BlockSpec((NB, 1), lambda i: (i, 0)),
            pl.BlockSpec((NB, 1), lambda i: (i, 0)),
        ],
        out_shape=[
            jax.ShapeDtypeStruct((N, H), jnp.float32),
            jax.ShapeDtypeStruct((N, 1), jnp.float32),
            jax.ShapeDtypeStruct((N, 1), jnp.float32),
        ],
    )(deg, xin)


def _layer_body(p_ref, ni_ref, no_ref, w_ref, b_ref,
                z_ref, rs_ref, rm_ref, sacc, macc):
    i = pl.program_id(0)
    agg = (p_ref[0] + p_ref[1]) * ni_ref[...]
    x = jnp.dot(agg, w_ref[...], preferred_element_type=jnp.float32, precision=lax.Precision.HIGHEST)
    x = jnp.maximum(x + b_ref[...], 0.0)
    z_ref[...] = x * no_ref[...]
    ps = jnp.sum(x, axis=0, keepdims=True)
    pm = jnp.max(x, axis=0, keepdims=True)

    @pl.when(i == 0)
    def _():
        sacc[...] = ps
        macc[...] = pm

    @pl.when(i > 0)
    def _():
        sacc[...] += ps
        macc[...] = jnp.maximum(macc[...], pm)

    @pl.when(i == pl.num_programs(0) - 1)
    def _():
        rs_ref[...] = sacc[...]
        rm_ref[...] = macc[...]


def _layer_call(partials, norm_in, norm_out, w, b):
    return pl.pallas_call(
        _layer_body,
        grid=(GRID,),
        in_specs=[
            pl.BlockSpec((2, NB, H), lambda i: (0, i, 0)),
            pl.BlockSpec((NB, 1), lambda i: (i, 0)),
            pl.BlockSpec((NB, 1), lambda i: (i, 0)),
            pl.BlockSpec((H, H), lambda i: (0, 0)),
            pl.BlockSpec((1, H), lambda i: (0, 0)),
        ],
        out_specs=[
            pl.BlockSpec((NB, H), lambda i: (i, 0)),
            pl.BlockSpec((1, H), lambda i: (0, 0)),
            pl.BlockSpec((1, H), lambda i: (0, 0)),
        ],
        out_shape=[
            jax.ShapeDtypeStruct((N, H), jnp.float32),
            jax.ShapeDtypeStruct((1, H), jnp.float32),
            jax.ShapeDtypeStruct((1, H), jnp.float32),
        ],
        scratch_shapes=[
            pltpu.VMEM((1, H), jnp.float32),
            pltpu.VMEM((1, H), jnp.float32),
        ],
    )(partials, norm_in, norm_out, w, b)


def _head_body(inpc_ref, wc_ref, rs1, rm1, rs2, rm2, rs3, rm3,
               w1c_ref, w1s_ref, w1m_ref, bm1_ref, wm2_ref, bm2_ref, wm3_ref,
               out_ref):
    embed = jnp.maximum(
        jnp.dot(inpc_ref[...], wc_ref[...], preferred_element_type=jnp.float32, precision=lax.Precision.HIGHEST),
        0.0)
    inv_n = 1.0 / N
    hs = (jnp.maximum(rs1[...] * inv_n, 0.0)
          + jnp.maximum(rs2[...] * inv_n, 0.0)
          + jnp.maximum(rs3[...] * inv_n, 0.0))
    hm = (jnp.maximum(rm1[...], 0.0) + jnp.maximum(rm2[...], 0.0)
          + jnp.maximum(rm3[...], 0.0))
    h = (jnp.dot(embed, w1c_ref[...], preferred_element_type=jnp.float32, precision=lax.Precision.HIGHEST)
         + jnp.dot(hs, w1s_ref[...], preferred_element_type=jnp.float32, precision=lax.Precision.HIGHEST)
         + jnp.dot(hm, w1m_ref[...], preferred_element_type=jnp.float32, precision=lax.Precision.HIGHEST)
         + bm1_ref[...])
    h = jnp.maximum(h, 0.0)
    h = jnp.maximum(
        jnp.dot(h, wm2_ref[...], preferred_element_type=jnp.float32, precision=lax.Precision.HIGHEST)
        + bm2_ref[...], 0.0)
    out_ref[...] = jnp.dot(h, wm3_ref[...], preferred_element_type=jnp.float32, precision=lax.Precision.HIGHEST)


def _head_call(inp_c, wc, rs1, rm1, rs2, rm2, rs3, rm3,
               w1c, w1s, w1m, bm1, wm2, bm2, wm3):
    return pl.pallas_call(
        _head_body,
        out_shape=jax.ShapeDtypeStruct((1, 1), jnp.float32),
    )(inp_c, wc, rs1, rm1, rs2, rm2, rs3, rm3,
      w1c, w1s, w1m, bm1, wm2, bm2, wm3)


def kernel(inp_c, inp_n, edge_index, W1, b1, W2, b2, W3, b3,
           Wc, Wm1, bm1, Wm2, bm2, Wm3):
    src = edge_index[0]
    dst = edge_index[1]
    pad = EPAD - E
    src_p = jnp.concatenate(
        [src, jnp.zeros((pad,), jnp.int32)]).reshape(NW, NCHUNK, CH)
    dst_p = jnp.concatenate(
        [dst, jnp.full((pad,), N, jnp.int32)]).reshape(NW, NCHUNK, CH)

    col = jnp.arange(H, dtype=jnp.float32)
    ones_src = jnp.tile(jnp.where(col < 64, 1.0, 0.0)[None, :], (CH, 1))
    ones_dst = jnp.tile(jnp.where(col < 64, 0.0, 1.0)[None, :], (CH, 1))
    zeros_h = jnp.zeros((RPS, H), jnp.float32)

    deg = _deg_kernel(src_p, dst_p, ones_src, ones_dst, zeros_h)
    z, norm_out, norm_in = _prep_call(deg, inp_n)

    readouts = []
    for w, b in ((W1, b1), (W2, b2), (W3, b3)):
        partials = _spmm_kernel(z, src_p, dst_p, zeros_h)
        z, rs, rm = _layer_call(partials, norm_in, norm_out,
                                w, b.reshape(1, H))
        readouts += [rs, rm]

    w1c = Wm1[:64]
    w1s = Wm1[64:64 + H]
    w1m = Wm1[64 + H:]
    return _head_call(inp_c, Wc, *readouts,
                      w1c, w1s, w1m, bm1.reshape(1, -1),
                      Wm2, bm2.reshape(1, -1), Wm3)
